# baseline (speedup 1.0000x reference)
import functools
import math

import jax
import jax.numpy as jnp
from jax import lax
from jax.experimental import pallas as pl
from jax.experimental.pallas import tpu as pltpu
from jax.experimental.pallas import tpu_sc as plsc

N0 = 10000
E0 = 160000
F_IN = 256
HID = 1024
F_OUT = 256

N0P = 10240
K1 = 5000
N1P = 5120
K2 = 2500
N2P = 2560
EP = 163840
NC, NS = 2, 16
LANES = 16

_mesh = plsc.VectorSubcoreMesh(core_axis_name="c", subcore_axis_name="s")
_sc_params = pltpu.CompilerParams(needs_layout_passes=False)

INT_MIN = -2147483648


def _sortable_i32(f):
    b = lax.bitcast_convert_type(f, jnp.int32)
    return jnp.where(b >= 0, b,
                     jnp.bitwise_xor(jnp.invert(b), jnp.int32(INT_MIN)))


@functools.lru_cache(None)
def _sc_segsum(n_p, ep, c_total, w):
    B = 128
    ept = ep // NS
    nblk = ept // B
    nrt = n_p // NS
    cpc = c_total // NC

    def body(u_hbm, src_hbm, dst_hbm, out_hbm, srcv, dstv, idxv, rows, zbuf,
             acc, sem):
        cid = lax.axis_index("c")
        sid = lax.axis_index("s")
        for r in range(16):
            for q in range(w // 16):
                zbuf[r, pl.ds(q * 16, 16)] = jnp.zeros((16,), jnp.float32)
        ebase = sid * ept
        for cc in range(cpc):
            c = cid * cpc + cc
            for i in range(nrt // 16):
                pltpu.sync_copy(zbuf, acc.at[pl.ds(sid * nrt + i * 16, 16), :])
            plsc.subcore_barrier()

            def step(j, carry):
                pltpu.sync_copy(src_hbm.at[pl.ds(ebase + j * B, B)], srcv)
                pltpu.sync_copy(dst_hbm.at[pl.ds(ebase + j * B, B)], dstv)
                for q in range(B // 16):
                    s16 = srcv[pl.ds(q * 16, 16)]
                    idxv[pl.ds(q * 16, 16)] = s16 * c_total + c
                pltpu.async_copy(u_hbm.at[idxv], rows, sem).wait()
                pltpu.sync_copy(rows, acc.at[dstv], add=True)
                return carry

            lax.fori_loop(0, nblk, step, jnp.int32(0))
            plsc.subcore_barrier()
            pltpu.sync_copy(acc.at[pl.ds(sid * nrt, nrt), :],
                            out_hbm.at[c, pl.ds(sid * nrt, nrt), :])

    return functools.partial(
        pl.kernel, body,
        out_type=jax.ShapeDtypeStruct((c_total, n_p, w), jnp.float32),
        mesh=_mesh,
        scratch_types=[
            pltpu.VMEM((B,), jnp.int32),
            pltpu.VMEM((B,), jnp.int32),
            pltpu.VMEM((B,), jnp.int32),
            pltpu.VMEM((B, w), jnp.float32),
            pltpu.VMEM((16, w), jnp.float32),
            pltpu.VMEM_SHARED((n_p, w), jnp.float32),
            pltpu.SemaphoreType.DMA,
        ],
    )()


def _hist_reduce_write(shared, hred, outv, out_hbm, cid, sid, n_new_p):
    nch = n_new_p // 128
    per = (nch + NS - 1) // NS
    plsc.subcore_barrier()
    for k in range(per):
        idx = sid + k * NS

        @pl.when(idx < nch)
        def _():
            off = pl.multiple_of(idx * 128, 128)
            pltpu.sync_copy(shared.at[:, pl.ds(off, 128)], hred)
            for g in range(8):
                acc = jnp.zeros((16,), jnp.float32)
                for t in range(NS):
                    acc = acc + hred[t, pl.ds(g * 16, 16)]
                outv[pl.ds(g * 16, 16)] = acc
            oof = pl.multiple_of(cid * n_new_p + idx * 128, 128)
            pltpu.sync_copy(outv, out_hbm.at[pl.ds(oof, 128)])


@functools.lru_cache(None)
def _sc_deg0(n_p, ep):
    ept = ep // (NC * NS)
    B = 512

    def body(dst_hbm, out_hbm, dstv, hist, hred, outv, shared, sem):
        cid = lax.axis_index("c")
        sid = lax.axis_index("s")
        for j in range(n_p // 16):
            hist[pl.ds(j * 16, 16)] = jnp.zeros((16,), jnp.float32)
        ones = jnp.ones((16,), jnp.float32)
        wid = cid * NS + sid
        ebase = wid * ept

        def step(j, carry):
            pltpu.sync_copy(dst_hbm.at[pl.ds(ebase + j * B, B)], dstv)
            for q in range(B // 16):
                d16 = dstv[pl.ds(q * 16, 16)]
                plsc.addupdate_scatter(hist, [d16], ones, mask=d16 >= 0)
            return carry

        lax.fori_loop(0, ept // B, step, jnp.int32(0))
        pltpu.sync_copy(hist, shared.at[sid])
        _hist_reduce_write(shared, hred, outv, out_hbm, cid, sid, n_p)

    return functools.partial(
        pl.kernel, body,
        out_type=jax.ShapeDtypeStruct((2 * n_p,), jnp.float32),
        mesh=_mesh,
        compiler_params=_sc_params,
        scratch_types=[
            pltpu.VMEM((B,), jnp.int32),
            pltpu.VMEM((n_p,), jnp.float32),
            pltpu.VMEM((NS, 128), jnp.float32),
            pltpu.VMEM((128,), jnp.float32),
            pltpu.VMEM_SHARED((NS, n_p), jnp.float32),
            pltpu.SemaphoreType.DMA,
        ],
    )()


@functools.lru_cache(None)
def _sc_relabel(n_old_p, n_new_p, n_new, ep):
    ept = ep // (NC * NS)
    B = 512

    def body(src_hbm, dst_hbm, nidx_hbm, ns_hbm, nd_hbm, deg_hbm,
             srcv, dstv, nsv, ndv, tbl, hist, hred, outv, shared, sem):
        cid = lax.axis_index("c")
        sid = lax.axis_index("s")
        pltpu.sync_copy(nidx_hbm, tbl)
        for j in range(n_new_p // 16):
            hist[pl.ds(j * 16, 16)] = jnp.zeros((16,), jnp.float32)
        ones = jnp.ones((16,), jnp.float32)
        wid = cid * NS + sid
        ebase = wid * ept

        def step(j, carry):
            pltpu.sync_copy(src_hbm.at[pl.ds(ebase + j * B, B)], srcv)
            pltpu.sync_copy(dst_hbm.at[pl.ds(ebase + j * B, B)], dstv)
            for q in range(B // 16):
                s16 = srcv[pl.ds(q * 16, 16)]
                d16 = dstv[pl.ds(q * 16, 16)]
                ns16 = plsc.load_gather(tbl, [s16])
                nd16 = plsc.load_gather(tbl, [d16])
                valid = jnp.logical_and(ns16 >= 0, nd16 >= 0)
                nsv[pl.ds(q * 16, 16)] = jnp.where(valid, ns16, 0)
                ndq = jnp.where(valid, nd16, jnp.int32(n_new))
                ndv[pl.ds(q * 16, 16)] = ndq
                plsc.addupdate_scatter(hist, [ndq], ones, mask=valid)
            pltpu.sync_copy(nsv, ns_hbm.at[pl.ds(ebase + j * B, B)])
            pltpu.sync_copy(ndv, nd_hbm.at[pl.ds(ebase + j * B, B)])
            return carry

        lax.fori_loop(0, ept // B, step, jnp.int32(0))
        pltpu.sync_copy(hist, shared.at[sid])
        _hist_reduce_write(shared, hred, outv, deg_hbm, cid, sid, n_new_p)

    return functools.partial(
        pl.kernel, body,
        out_type=[jax.ShapeDtypeStruct((ep,), jnp.int32),
                  jax.ShapeDtypeStruct((ep,), jnp.int32),
                  jax.ShapeDtypeStruct((2 * n_new_p,), jnp.float32)],
        mesh=_mesh,
        compiler_params=_sc_params,
        scratch_types=[
            pltpu.VMEM((B,), jnp.int32),
            pltpu.VMEM((B,), jnp.int32),
            pltpu.VMEM((B,), jnp.int32),
            pltpu.VMEM((B,), jnp.int32),
            pltpu.VMEM((n_old_p,), jnp.int32),
            pltpu.VMEM((n_new_p,), jnp.float32),
            pltpu.VMEM((NS, 128), jnp.float32),
            pltpu.VMEM((128,), jnp.float32),
            pltpu.VMEM_SHARED((NS, n_new_p), jnp.float32),
            pltpu.SemaphoreType.DMA,
        ],
    )()


@functools.lru_cache(None)
def _sc_compact(n_p, k_p, n_real, k_real):
    npt = n_p // NS
    kcols = k_p // NS

    def body(score_hbm, thr_hbm, perm_hbm, nidx_hbm,
             scv, thrbuf, cbuf, ccopy, permtile, nidxv, pred, poutv,
             shared_cnt, shared_perm, sem):
        cid = lax.axis_index("c")
        sid = lax.axis_index("s")

        @pl.when(cid == 0)
        def _():
            iot = lax.iota(jnp.int32, 16)
            pltpu.sync_copy(score_hbm.at[pl.ds(sid * npt, npt)], scv)
            pltpu.sync_copy(thr_hbm, thrbuf)
            thr = jnp.sum(jnp.where(iot == 0, thrbuf[0, pl.ds(0, 16)], 0))
            ties = jnp.sum(jnp.where(iot == 0, thrbuf[1, pl.ds(0, 16)], 0))

            def key_at(j):
                s16 = scv[pl.ds(j * 16, 16)]
                k16 = _sortable_i32(s16)
                glob = sid * npt + j * 16 + iot
                return jnp.where(glob < n_real, k16, INT_MIN), glob

            cnt_gt = jnp.int32(0)
            cnt_eq = jnp.int32(0)
            for j in range(npt // 16):
                k16, _ = key_at(j)
                cnt_gt = cnt_gt + plsc.all_reduce_population_count(k16 > thr)[0]
                cnt_eq = cnt_eq + plsc.all_reduce_population_count(k16 == thr)[0]
            cbuf[...] = (jnp.where(iot == 0, cnt_gt, 0)
                         + jnp.where(iot == 1, cnt_eq, 0))
            pltpu.sync_copy(cbuf, shared_cnt.at[sid])
            plsc.subcore_barrier()
            pltpu.sync_copy(shared_cnt, ccopy)
            gts = plsc.load_gather(ccopy, [iot, jnp.zeros((16,), jnp.int32)])
            eqs = plsc.load_gather(ccopy, [iot, jnp.ones((16,), jnp.int32)])
            base_gt = jnp.sum(jnp.where(iot < sid, gts, 0))
            base_eq = jnp.sum(jnp.where(iot < sid, eqs, 0))

            for j in range(k_p // 16):
                permtile[pl.ds(j * 16, 16)] = jnp.zeros((16,), jnp.int32)
            rg = base_gt
            re = base_eq
            for j in range(npt // 16):
                k16, glob = key_at(j)
                sgt = k16 > thr
                seq = k16 == thr
                igt = jnp.where(sgt, 1, 0)
                ieq = jnp.where(seq, 1, 0)
                egt = rg + plsc.cumsum(igt) - igt
                eeq = re + plsc.cumsum(ieq) - ieq
                sel = jnp.logical_or(sgt, jnp.logical_and(seq, eeq < ties))
                pos = egt + jnp.minimum(eeq, ties)
                plsc.store_scatter(permtile, [pos], glob, mask=sel)
                nidxv[pl.ds(j * 16, 16)] = jnp.where(sel, pos, -1)
                rg = rg + plsc.all_reduce_population_count(sgt)[0]
                re = re + plsc.all_reduce_population_count(seq)[0]
            pltpu.sync_copy(nidxv, nidx_hbm.at[pl.ds(sid * npt, npt)])
            pltpu.sync_copy(permtile, shared_perm.at[sid])
            plsc.subcore_barrier()
            kch = k_p // 128
            for k in range((kch + NS - 1) // NS):
                idx = sid + k * NS

                @pl.when(idx < kch)
                def _():
                    off = pl.multiple_of(idx * 128, 128)
                    pltpu.sync_copy(shared_perm.at[:, pl.ds(off, 128)], pred)
                    for g in range(8):
                        acc = jnp.zeros((16,), jnp.int32)
                        for t in range(NS):
                            acc = acc + pred[t, pl.ds(g * 16, 16)]
                        poutv[pl.ds(g * 16, 16)] = acc
                    pltpu.sync_copy(poutv, perm_hbm.at[pl.ds(off, 128)])

    return functools.partial(
        pl.kernel, body,
        out_type=[jax.ShapeDtypeStruct((k_p,), jnp.int32),
                  jax.ShapeDtypeStruct((n_p,), jnp.int32)],
        mesh=_mesh,
        compiler_params=_sc_params,
        scratch_types=[
            pltpu.VMEM((npt,), jnp.float32),
            pltpu.VMEM((8, 128), jnp.int32),
            pltpu.VMEM((16,), jnp.int32),
            pltpu.VMEM((NS, 16), jnp.int32),
            pltpu.VMEM((k_p,), jnp.int32),
            pltpu.VMEM((npt,), jnp.int32),
            pltpu.VMEM((NS, 128), jnp.int32),
            pltpu.VMEM((128,), jnp.int32),
            pltpu.VMEM_SHARED((NS, 16), jnp.int32),
            pltpu.VMEM_SHARED((NS, k_p), jnp.int32),
            pltpu.SemaphoreType.DMA,
        ],
    )()


@functools.lru_cache(None)
def _sc_gather_rows(n_out_p, n_src_p, f, with_vals, clamp):
    rpt = n_out_p // (NC * NS)
    B = 32 if rpt % 32 == 0 else 16
    assert rpt % B == 0

    def body(*args):
        if with_vals:
            (x_hbm, idx_hbm, sc_hbm, out_hbm, vals_hbm,
             idxv, rows, sctbl, valsv, sem) = args
        else:
            x_hbm, idx_hbm, out_hbm, idxv, rows, sem = args
        cid = lax.axis_index("c")
        sid = lax.axis_index("s")
        wid = cid * NS + sid
        base = wid * rpt
        if with_vals:
            pltpu.sync_copy(sc_hbm, sctbl)

        def step(j, carry):
            off = base + j * B
            pltpu.sync_copy(idx_hbm.at[pl.ds(off, B)], idxv)
            if clamp:
                for q in range(B // 16):
                    i16 = idxv[pl.ds(q * 16, 16)]
                    idxv[pl.ds(q * 16, 16)] = jnp.maximum(i16, 0)
            pltpu.async_copy(x_hbm.at[idxv], rows, sem).wait()
            pltpu.sync_copy(rows, out_hbm.at[pl.ds(off, B), :])
            if with_vals:
                for q in range(B // 16):
                    i16 = idxv[pl.ds(q * 16, 16)]
                    valsv[pl.ds(q * 16, 16)] = plsc.load_gather(sctbl, [i16])
                pltpu.sync_copy(valsv, vals_hbm.at[pl.ds(off, B)])
            return carry

        lax.fori_loop(0, rpt // B, step, jnp.int32(0))

    out_type = [jax.ShapeDtypeStruct((n_out_p, f), jnp.float32)]
    scratch = [pltpu.VMEM((B,), jnp.int32),
               pltpu.VMEM((B, f), jnp.float32)]
    if with_vals:
        out_type.append(jax.ShapeDtypeStruct((n_out_p,), jnp.float32))
        scratch += [pltpu.VMEM((n_src_p,), jnp.float32),
                    pltpu.VMEM((B,), jnp.float32)]
    scratch.append(pltpu.SemaphoreType.DMA)

    return functools.partial(
        pl.kernel, body,
        out_type=out_type if with_vals else out_type[0],
        mesh=_mesh,
        compiler_params=_sc_params,
        scratch_types=scratch,
    )()


def _combine_mm_body(*refs, c_total, relu, with_score, with_mm, tanh_dinv):
    refs = list(refs)
    vch = refs.pop(0) if c_total > 0 else None
    u = refs.pop(0)
    dinv = refs.pop(0)
    w = refs.pop(0) if with_mm else None
    b = refs.pop(0)
    p = refs.pop(0) if with_score else None
    y = refs.pop(0)
    sc = refs.pop(0) if with_score else None
    dv = jnp.tanh(dinv[...]) if tanh_dinv else dinv[...]
    if c_total > 0:
        parts = [vch[c] for c in range(c_total)]
        v = parts[0] if c_total == 1 else jnp.concatenate(parts, axis=-1)
        a = (v + u[...]) * dv
    else:
        a = u[...] * dv
    if with_mm:
        acc = jnp.dot(a, w[...], preferred_element_type=jnp.float32) + b[...]
    else:
        acc = a + b[...]
    if relu:
        acc = jnp.maximum(acc, 0.0)
    y[...] = acc
    if with_score:
        pv = p[...]
        nrm = jnp.sqrt(jnp.sum(pv * pv))
        sc[...] = jnp.dot(acc, pv, preferred_element_type=jnp.float32) / nrm


@functools.lru_cache(None)
def _tc_combine_mm(n_p, c_total, w_chunk, f_in, f_out, relu, with_score,
                   with_mm, tanh_dinv=False, bm=512):
    grid = (n_p // bm,)
    in_specs = []
    if c_total > 0:
        in_specs.append(pl.BlockSpec((c_total, bm, w_chunk),
                                     lambda i: (0, i, 0)))
    in_specs.append(pl.BlockSpec((bm, f_in), lambda i: (i, 0)))
    in_specs.append(pl.BlockSpec((bm, 1), lambda i: (i, 0)))
    if with_mm:
        in_specs.append(pl.BlockSpec((f_in, f_out), lambda i: (0, 0)))
    in_specs.append(pl.BlockSpec((1, f_out), lambda i: (0, 0)))
    out_specs = pl.BlockSpec((bm, f_out), lambda i: (i, 0))
    out_shape = jax.ShapeDtypeStruct((n_p, f_out), jnp.float32)
    if with_score:
        in_specs.append(pl.BlockSpec((f_out, 1), lambda i: (0, 0)))
        out_specs = [out_specs, pl.BlockSpec((bm, 1), lambda i: (i, 0))]
        out_shape = [out_shape, jax.ShapeDtypeStruct((n_p, 1), jnp.float32)]
    body = functools.partial(_combine_mm_body, c_total=c_total, relu=relu,
                             with_score=with_score, with_mm=with_mm,
                             tanh_dinv=tanh_dinv)
    return pl.pallas_call(body, grid=grid, in_specs=in_specs,
                          out_specs=out_specs, out_shape=out_shape)


def _threshold_body(sc_ref, o_ref, *, k_sel, n_real, rows):
    s = sc_ref[...]
    key = _sortable_i32(s)
    flat = (lax.broadcasted_iota(jnp.int32, (rows, 128), 0) * 128
            + lax.broadcasted_iota(jnp.int32, (rows, 128), 1))
    key = jnp.where(flat < n_real, key, INT_MIN)
    sign = INT_MIN

    def step(i, t_u):
        kbit = 31 - i
        trial = jnp.bitwise_or(t_u, jnp.left_shift(jnp.int32(1), kbit))
        t_s = jnp.bitwise_xor(trial, sign)
        cnt = jnp.sum((key >= t_s).astype(jnp.int32))
        return jnp.where(cnt >= k_sel, trial, t_u)

    t_u = lax.fori_loop(0, 32, step, jnp.int32(0))
    t_s = jnp.bitwise_xor(t_u, sign)
    cnt_gt = jnp.sum((key > t_s).astype(jnp.int32))
    ties = k_sel - cnt_gt
    row = lax.broadcasted_iota(jnp.int32, (8, 128), 0)
    o_ref[...] = jnp.where(row == 0, t_s, jnp.where(row == 1, ties, 0))


@functools.lru_cache(None)
def _tc_threshold(n_p, k_sel, n_real):
    rows = n_p // 128
    body = functools.partial(_threshold_body, k_sel=k_sel, n_real=n_real,
                             rows=rows)
    return pl.pallas_call(
        body,
        in_specs=[pl.BlockSpec((rows, 128), lambda: (0, 0))],
        out_specs=pl.BlockSpec((8, 128), lambda: (0, 0)),
        out_shape=jax.ShapeDtypeStruct((8, 128), jnp.int32),
        grid=(),
    )


def _uscale_body(*refs, mode):
    if mode == "plain":
        x, dinv, o = refs
        o[...] = x[...] * dinv[...]
    elif mode == "tanh":
        x, dinv, vals, o = refs
        o[...] = x[...] * (dinv[...] * jnp.tanh(vals[...]))
    else:
        x, dinv, nidx, o = refs
        m = (nidx[...] >= 0).astype(jnp.float32)
        o[...] = x[...] * (dinv[...] * m)


@functools.lru_cache(None)
def _tc_uscale(n_p, f, mode, bm=512):
    assert n_p % bm == 0
    grid = (n_p // bm,)
    in_specs = [pl.BlockSpec((bm, f), lambda i: (i, 0)),
                pl.BlockSpec((bm, 1), lambda i: (i, 0))]
    if mode == "tanh":
        in_specs.append(pl.BlockSpec((bm, 1), lambda i: (i, 0)))
    elif mode == "mask":
        in_specs.append(pl.BlockSpec((bm, 1), lambda i: (i, 0)))
    return pl.pallas_call(
        functools.partial(_uscale_body, mode=mode),
        grid=grid, in_specs=in_specs,
        out_specs=pl.BlockSpec((bm, f), lambda i: (i, 0)),
        out_shape=jax.ShapeDtypeStruct((n_p, f), jnp.float32),
    )


def _dinv_col(deg2, n_p):
    return lax.rsqrt(deg2[:n_p] + deg2[n_p:] + 1.0).reshape(-1, 1)


def _segsum(u, src, dst, n_p, c_total, w):
    u2d = u.reshape(n_p * c_total, w)
    return _sc_segsum(n_p, EP, c_total, w)(u2d, src, dst)


def _pool(x_lvl, score, src, dst, n_p, k_p, n_real, k_real, n_old_p):
    sc_flat = score.reshape(n_p)
    thr = _tc_threshold(n_p, k_real, n_real)(score.reshape(n_p // 128, 128))
    perm, nidx = _sc_compact(n_p, k_p, n_real, k_real)(sc_flat, thr)
    ns, nd, deg2 = _sc_relabel(n_p, k_p, k_real, EP)(src, dst, nidx)
    xp, vals = _sc_gather_rows(k_p, n_p, HID, True, False)(x_lvl, perm, sc_flat)
    return xp, vals, ns, nd, _dinv_col(deg2, k_p), nidx


def kernel(x, edge_index, batch, W1, b1, W2, b2, W3, b3, W4, b4, W5, b5,
           W6, b6, p1, p2):
    src0 = jnp.concatenate([edge_index[0].astype(jnp.int32),
                            jnp.zeros((EP - E0,), jnp.int32)])
    dst0 = jnp.concatenate([edge_index[1].astype(jnp.int32),
                            jnp.full((EP - E0,), N0, jnp.int32)])
    xp = jnp.concatenate([x, jnp.zeros((N0P - N0, F_IN), jnp.float32)])

    deg0 = _sc_deg0(N0P, EP)(dst0)
    dinv0 = _dinv_col(deg0, N0P)

    ones0 = jnp.ones((N0P, 1), jnp.float32)
    zbh = jnp.zeros((1, HID), jnp.float32)

    xw1 = _tc_combine_mm(N0P, 0, 0, F_IN, HID, False, False, True)(
        xp, ones0, W1, zbh)
    u0 = _tc_uscale(N0P, HID, "plain")(xw1, dinv0)
    v0 = _segsum(u0, src0, dst0, N0P, 8, 128)
    x1, s1 = _tc_combine_mm(N0P, 8, 128, HID, HID, True, True, False)(
        v0, u0, dinv0, b1.reshape(1, HID), p1.reshape(HID, 1))

    x1p, vals1, ns1, nd1, dinv1, nidx1 = _pool(
        x1, s1, src0, dst0, N0P, N1P, N0, K1, N0P)

    xw2 = _tc_combine_mm(N1P, 0, 0, HID, HID, False, False, True,
                         tanh_dinv=True)(x1p, vals1.reshape(N1P, 1), W2, zbh)
    u1 = _tc_uscale(N1P, HID, "plain")(xw2, dinv1)
    v1 = _segsum(u1, ns1, nd1, N1P, 8, 128)
    x2, s2 = _tc_combine_mm(N1P, 8, 128, HID, HID, True, True, False)(
        v1, u1, dinv1, b2.reshape(1, HID), p2.reshape(HID, 1))

    x2p, vals2, ns2, nd2, dinv2, nidx2 = _pool(
        x2, s2, ns1, nd1, N1P, N2P, K1, K2, N1P)

    u2 = _tc_uscale(N2P, HID, "tanh")(x2p, dinv2, vals2.reshape(N2P, 1))
    v2 = _segsum(u2, ns2, nd2, N2P, 8, 128)
    x3 = _tc_combine_mm(N2P, 8, 128, HID, HID, True, False, True)(
        v2, u2, dinv2, W3, b3.reshape(1, HID))

    h1 = _sc_gather_rows(N1P, N2P, HID, False, True)(x3, nidx2)
    u1b = _tc_uscale(N1P, HID, "mask")(h1, dinv1, nidx2.reshape(N1P, 1))
    v1b = _segsum(u1b, ns1, nd1, N1P, 8, 128)
    x4 = _tc_combine_mm(N1P, 8, 128, HID, HID, True, False, True)(
        v1b, u1b, dinv1, W4, b4.reshape(1, HID))

    h0 = _sc_gather_rows(N0P, N1P, HID, False, True)(x4, nidx1)
    u0b = _tc_uscale(N0P, HID, "mask")(h0, dinv0, nidx1.reshape(N0P, 1))
    v0b = _segsum(u0b, src0, dst0, N0P, 8, 128)
    x5 = _tc_combine_mm(N0P, 8, 128, HID, HID, True, False, True)(
        v0b, u0b, dinv0, W5, b5.reshape(1, HID))

    u5 = _tc_uscale(N0P, HID, "plain")(x5, dinv0)
    zb = jnp.zeros((1, F_OUT), jnp.float32)
    w6 = _tc_combine_mm(N0P, 0, 0, HID, F_OUT, False, False, True)(
        u5, dinv0 * 0.0 + 1.0, W6, zb)
    v6 = _segsum(w6, src0, dst0, N0P, 2, 128)
    out = _tc_combine_mm(N0P, 2, 128, F_OUT, F_OUT, False, False, False)(
        v6, w6, dinv0, b6.reshape(1, F_OUT))
    return out[:N0]


# speedup vs baseline: 1.0127x; 1.0127x over previous
import functools
import math

import jax
import jax.numpy as jnp
from jax import lax
from jax.experimental import pallas as pl
from jax.experimental.pallas import tpu as pltpu
from jax.experimental.pallas import tpu_sc as plsc

N0 = 10000
E0 = 160000
F_IN = 256
HID = 1024
F_OUT = 256

N0P = 10240
K1 = 5000
N1P = 5120
K2 = 2500
N2P = 2560
EP = 163840
NC, NS = 2, 16
LANES = 16

_mesh = plsc.VectorSubcoreMesh(core_axis_name="c", subcore_axis_name="s")
_sc_params = pltpu.CompilerParams(needs_layout_passes=False)

INT_MIN = -2147483648


def _sortable_i32(f):
    b = lax.bitcast_convert_type(f, jnp.int32)
    return jnp.where(b >= 0, b,
                     jnp.bitwise_xor(jnp.invert(b), jnp.int32(INT_MIN)))


@functools.lru_cache(None)
def _sc_segsum(n_p, ep, c_total, w):
    B = 128
    ept = ep // NS
    nblk = ept // B
    nrt = n_p // NS
    cpc = c_total // NC

    def body(u_hbm, src_hbm, dst2d_hbm, out_hbm, idxall, dstv2, rows, zbuf,
             acc, sem):
        cid = lax.axis_index("c")
        sid = lax.axis_index("s")
        for r in range(8):
            for q in range(w // 16):
                zbuf[r, pl.ds(q * 16, 16)] = jnp.zeros((16,), jnp.float32)
        ebase = sid * ept
        rbase = sid * (ept // B)
        for cc in range(cpc):
            c = cid * cpc + cc
            for i in range(nrt // 8):
                pltpu.sync_copy(zbuf, acc.at[pl.ds(sid * nrt + i * 8, 8), :])
            pltpu.sync_copy(src_hbm.at[pl.ds(ebase, ept)], idxall)

            def cvt(q, carry):
                s16 = idxall[pl.ds(q * 16, 16)]
                idxall[pl.ds(q * 16, 16)] = s16 * c_total + c
                return carry

            lax.fori_loop(0, ept // 16, cvt, jnp.int32(0))
            plsc.subcore_barrier()
            pltpu.sync_copy(dst2d_hbm.at[pl.ds(rbase, 1), :], dstv2.at[pl.ds(0, 1), :])
            pltpu.async_copy(u_hbm.at[idxall.at[pl.ds(0, B)]], rows.at[0],
                             sem)

            def step(g, carry):
                b = lax.rem(g, 2)
                pltpu.make_async_copy(u_hbm.at[idxall.at[pl.ds(0, B)]],
                                      rows.at[b], sem).wait()

                @pl.when(g + 1 < nblk)
                def _():
                    off = pl.multiple_of((g + 1) * B, B)
                    pltpu.async_copy(u_hbm.at[idxall.at[pl.ds(off, B)]],
                                     rows.at[1 - b], sem)
                    pltpu.sync_copy(dst2d_hbm.at[pl.ds(rbase + g + 1, 1), :],
                                    dstv2.at[pl.ds(1 - b, 1), :])
                pltpu.sync_copy(rows.at[b], acc.at[dstv2.at[b]], add=True)
                return carry

            lax.fori_loop(0, nblk, step, jnp.int32(0))
            plsc.subcore_barrier()
            pltpu.sync_copy(acc.at[pl.ds(sid * nrt, nrt), :],
                            out_hbm.at[c, pl.ds(sid * nrt, nrt), :])

    return functools.partial(
        pl.kernel, body,
        out_type=jax.ShapeDtypeStruct((c_total, n_p, w), jnp.float32),
        mesh=_mesh,
        scratch_types=[
            pltpu.VMEM((ept,), jnp.int32),
            pltpu.VMEM((2, B), jnp.int32),
            pltpu.VMEM((2, B, w), jnp.float32),
            pltpu.VMEM((8, w), jnp.float32),
            pltpu.VMEM_SHARED((n_p, w), jnp.float32),
            pltpu.SemaphoreType.DMA,
        ],
    )()


def _hist_reduce_write(shared, hred, outv, out_hbm, cid, sid, n_new_p):
    nch = n_new_p // 128
    per = (nch + NS - 1) // NS
    plsc.subcore_barrier()
    for k in range(per):
        idx = sid + k * NS

        @pl.when(idx < nch)
        def _():
            off = pl.multiple_of(idx * 128, 128)
            pltpu.sync_copy(shared.at[:, pl.ds(off, 128)], hred)
            for g in range(8):
                acc = jnp.zeros((16,), jnp.float32)
                for t in range(NS):
                    acc = acc + hred[t, pl.ds(g * 16, 16)]
                outv[pl.ds(g * 16, 16)] = acc
            oof = pl.multiple_of(cid * n_new_p + idx * 128, 128)
            pltpu.sync_copy(outv, out_hbm.at[pl.ds(oof, 128)])


@functools.lru_cache(None)
def _sc_deg0(n_p, ep):
    ept = ep // (NC * NS)
    B = 512

    def body(dst_hbm, out_hbm, dstv, hist, hred, outv, shared, sem):
        cid = lax.axis_index("c")
        sid = lax.axis_index("s")
        for j in range(n_p // 16):
            hist[pl.ds(j * 16, 16)] = jnp.zeros((16,), jnp.float32)
        ones = jnp.ones((16,), jnp.float32)
        wid = cid * NS + sid
        ebase = wid * ept

        def step(j, carry):
            pltpu.sync_copy(dst_hbm.at[pl.ds(ebase + j * B, B)], dstv)
            for q in range(B // 16):
                d16 = dstv[pl.ds(q * 16, 16)]
                plsc.addupdate_scatter(hist, [d16], ones, mask=d16 >= 0)
            return carry

        lax.fori_loop(0, ept // B, step, jnp.int32(0))
        pltpu.sync_copy(hist, shared.at[sid])
        _hist_reduce_write(shared, hred, outv, out_hbm, cid, sid, n_p)

    return functools.partial(
        pl.kernel, body,
        out_type=jax.ShapeDtypeStruct((2 * n_p,), jnp.float32),
        mesh=_mesh,
        compiler_params=_sc_params,
        scratch_types=[
            pltpu.VMEM((B,), jnp.int32),
            pltpu.VMEM((n_p,), jnp.float32),
            pltpu.VMEM((NS, 128), jnp.float32),
            pltpu.VMEM((128,), jnp.float32),
            pltpu.VMEM_SHARED((NS, n_p), jnp.float32),
            pltpu.SemaphoreType.DMA,
        ],
    )()


@functools.lru_cache(None)
def _sc_relabel(n_old_p, n_new_p, n_new, ep):
    ept = ep // (NC * NS)
    B = 512

    def body(src_hbm, dst_hbm, nidx_hbm, ns_hbm, nd_hbm, deg_hbm,
             srcv, dstv, nsv, ndv, tbl, hist, hred, outv, shared, sem):
        cid = lax.axis_index("c")
        sid = lax.axis_index("s")
        pltpu.sync_copy(nidx_hbm, tbl)
        for j in range(n_new_p // 16):
            hist[pl.ds(j * 16, 16)] = jnp.zeros((16,), jnp.float32)
        ones = jnp.ones((16,), jnp.float32)
        wid = cid * NS + sid
        ebase = wid * ept

        def step(j, carry):
            pltpu.sync_copy(src_hbm.at[pl.ds(ebase + j * B, B)], srcv)
            pltpu.sync_copy(dst_hbm.at[pl.ds(ebase + j * B, B)], dstv)
            for q in range(B // 16):
                s16 = srcv[pl.ds(q * 16, 16)]
                d16 = dstv[pl.ds(q * 16, 16)]
                ns16 = plsc.load_gather(tbl, [s16])
                nd16 = plsc.load_gather(tbl, [d16])
                valid = jnp.logical_and(ns16 >= 0, nd16 >= 0)
                nsv[pl.ds(q * 16, 16)] = jnp.where(valid, ns16, 0)
                ndq = jnp.where(valid, nd16, jnp.int32(n_new))
                ndv[pl.ds(q * 16, 16)] = ndq
                plsc.addupdate_scatter(hist, [ndq], ones, mask=valid)
            pltpu.sync_copy(nsv, ns_hbm.at[pl.ds(ebase + j * B, B)])
            pltpu.sync_copy(ndv, nd_hbm.at[pl.ds(ebase + j * B, B)])
            return carry

        lax.fori_loop(0, ept // B, step, jnp.int32(0))
        pltpu.sync_copy(hist, shared.at[sid])
        _hist_reduce_write(shared, hred, outv, deg_hbm, cid, sid, n_new_p)

    return functools.partial(
        pl.kernel, body,
        out_type=[jax.ShapeDtypeStruct((ep,), jnp.int32),
                  jax.ShapeDtypeStruct((ep,), jnp.int32),
                  jax.ShapeDtypeStruct((2 * n_new_p,), jnp.float32)],
        mesh=_mesh,
        compiler_params=_sc_params,
        scratch_types=[
            pltpu.VMEM((B,), jnp.int32),
            pltpu.VMEM((B,), jnp.int32),
            pltpu.VMEM((B,), jnp.int32),
            pltpu.VMEM((B,), jnp.int32),
            pltpu.VMEM((n_old_p,), jnp.int32),
            pltpu.VMEM((n_new_p,), jnp.float32),
            pltpu.VMEM((NS, 128), jnp.float32),
            pltpu.VMEM((128,), jnp.float32),
            pltpu.VMEM_SHARED((NS, n_new_p), jnp.float32),
            pltpu.SemaphoreType.DMA,
        ],
    )()


@functools.lru_cache(None)
def _sc_compact(n_p, k_p, n_real, k_real):
    npt = n_p // NS
    kcols = k_p // NS

    def body(score_hbm, thr_hbm, perm_hbm, nidx_hbm,
             scv, thrbuf, cbuf, ccopy, permtile, nidxv, pred, poutv,
             shared_cnt, shared_perm, sem):
        cid = lax.axis_index("c")
        sid = lax.axis_index("s")

        @pl.when(cid == 0)
        def _():
            iot = lax.iota(jnp.int32, 16)
            pltpu.sync_copy(score_hbm.at[pl.ds(sid * npt, npt)], scv)
            pltpu.sync_copy(thr_hbm, thrbuf)
            thr = jnp.sum(jnp.where(iot == 0, thrbuf[0, pl.ds(0, 16)], 0))
            ties = jnp.sum(jnp.where(iot == 0, thrbuf[1, pl.ds(0, 16)], 0))

            def key_at(j):
                s16 = scv[pl.ds(j * 16, 16)]
                k16 = _sortable_i32(s16)
                glob = sid * npt + j * 16 + iot
                return jnp.where(glob < n_real, k16, INT_MIN), glob

            cnt_gt = jnp.int32(0)
            cnt_eq = jnp.int32(0)
            for j in range(npt // 16):
                k16, _ = key_at(j)
                cnt_gt = cnt_gt + plsc.all_reduce_population_count(k16 > thr)[0]
                cnt_eq = cnt_eq + plsc.all_reduce_population_count(k16 == thr)[0]
            cbuf[...] = (jnp.where(iot == 0, cnt_gt, 0)
                         + jnp.where(iot == 1, cnt_eq, 0))
            pltpu.sync_copy(cbuf, shared_cnt.at[sid])
            plsc.subcore_barrier()
            pltpu.sync_copy(shared_cnt, ccopy)
            gts = plsc.load_gather(ccopy, [iot, jnp.zeros((16,), jnp.int32)])
            eqs = plsc.load_gather(ccopy, [iot, jnp.ones((16,), jnp.int32)])
            base_gt = jnp.sum(jnp.where(iot < sid, gts, 0))
            base_eq = jnp.sum(jnp.where(iot < sid, eqs, 0))

            for j in range(k_p // 16):
                permtile[pl.ds(j * 16, 16)] = jnp.zeros((16,), jnp.int32)
            rg = base_gt
            re = base_eq
            for j in range(npt // 16):
                k16, glob = key_at(j)
                sgt = k16 > thr
                seq = k16 == thr
                igt = jnp.where(sgt, 1, 0)
                ieq = jnp.where(seq, 1, 0)
                egt = rg + plsc.cumsum(igt) - igt
                eeq = re + plsc.cumsum(ieq) - ieq
                sel = jnp.logical_or(sgt, jnp.logical_and(seq, eeq < ties))
                pos = egt + jnp.minimum(eeq, ties)
                plsc.store_scatter(permtile, [pos], glob, mask=sel)
                nidxv[pl.ds(j * 16, 16)] = jnp.where(sel, pos, -1)
                rg = rg + plsc.all_reduce_population_count(sgt)[0]
                re = re + plsc.all_reduce_population_count(seq)[0]
            pltpu.sync_copy(nidxv, nidx_hbm.at[pl.ds(sid * npt, npt)])
            pltpu.sync_copy(permtile, shared_perm.at[sid])
            plsc.subcore_barrier()
            kch = k_p // 128
            for k in range((kch + NS - 1) // NS):
                idx = sid + k * NS

                @pl.when(idx < kch)
                def _():
                    off = pl.multiple_of(idx * 128, 128)
                    pltpu.sync_copy(shared_perm.at[:, pl.ds(off, 128)], pred)
                    for g in range(8):
                        acc = jnp.zeros((16,), jnp.int32)
                        for t in range(NS):
                            acc = acc + pred[t, pl.ds(g * 16, 16)]
                        poutv[pl.ds(g * 16, 16)] = acc
                    pltpu.sync_copy(poutv, perm_hbm.at[pl.ds(off, 128)])

    return functools.partial(
        pl.kernel, body,
        out_type=[jax.ShapeDtypeStruct((k_p,), jnp.int32),
                  jax.ShapeDtypeStruct((n_p,), jnp.int32)],
        mesh=_mesh,
        compiler_params=_sc_params,
        scratch_types=[
            pltpu.VMEM((npt,), jnp.float32),
            pltpu.VMEM((8, 128), jnp.int32),
            pltpu.VMEM((16,), jnp.int32),
            pltpu.VMEM((NS, 16), jnp.int32),
            pltpu.VMEM((k_p,), jnp.int32),
            pltpu.VMEM((npt,), jnp.int32),
            pltpu.VMEM((NS, 128), jnp.int32),
            pltpu.VMEM((128,), jnp.int32),
            pltpu.VMEM_SHARED((NS, 16), jnp.int32),
            pltpu.VMEM_SHARED((NS, k_p), jnp.int32),
            pltpu.SemaphoreType.DMA,
        ],
    )()


@functools.lru_cache(None)
def _sc_gather_rows(n_out_p, n_src_p, f, with_vals, clamp):
    rpt = n_out_p // (NC * NS)
    B = 32 if rpt % 32 == 0 else 16
    assert rpt % B == 0

    def body(*args):
        if with_vals:
            (x_hbm, idx_hbm, sc_hbm, out_hbm, vals_hbm,
             idxv, rows, sctbl, valsv, sem) = args
        else:
            x_hbm, idx_hbm, out_hbm, idxv, rows, sem = args
        cid = lax.axis_index("c")
        sid = lax.axis_index("s")
        wid = cid * NS + sid
        base = wid * rpt
        if with_vals:
            pltpu.sync_copy(sc_hbm, sctbl)

        def step(j, carry):
            off = base + j * B
            pltpu.sync_copy(idx_hbm.at[pl.ds(off, B)], idxv)
            if clamp:
                for q in range(B // 16):
                    i16 = idxv[pl.ds(q * 16, 16)]
                    idxv[pl.ds(q * 16, 16)] = jnp.maximum(i16, 0)
            pltpu.async_copy(x_hbm.at[idxv], rows, sem).wait()
            pltpu.sync_copy(rows, out_hbm.at[pl.ds(off, B), :])
            if with_vals:
                for q in range(B // 16):
                    i16 = idxv[pl.ds(q * 16, 16)]
                    valsv[pl.ds(q * 16, 16)] = plsc.load_gather(sctbl, [i16])
                pltpu.sync_copy(valsv, vals_hbm.at[pl.ds(off, B)])
            return carry

        lax.fori_loop(0, rpt // B, step, jnp.int32(0))

    out_type = [jax.ShapeDtypeStruct((n_out_p, f), jnp.float32)]
    scratch = [pltpu.VMEM((B,), jnp.int32),
               pltpu.VMEM((B, f), jnp.float32)]
    if with_vals:
        out_type.append(jax.ShapeDtypeStruct((n_out_p,), jnp.float32))
        scratch += [pltpu.VMEM((n_src_p,), jnp.float32),
                    pltpu.VMEM((B,), jnp.float32)]
    scratch.append(pltpu.SemaphoreType.DMA)

    return functools.partial(
        pl.kernel, body,
        out_type=out_type if with_vals else out_type[0],
        mesh=_mesh,
        compiler_params=_sc_params,
        scratch_types=scratch,
    )()


def _combine_mm_body(*refs, c_total, relu, with_score, with_mm, tanh_dinv):
    refs = list(refs)
    vch = refs.pop(0) if c_total > 0 else None
    u = refs.pop(0)
    dinv = refs.pop(0)
    w = refs.pop(0) if with_mm else None
    b = refs.pop(0)
    p = refs.pop(0) if with_score else None
    y = refs.pop(0)
    sc = refs.pop(0) if with_score else None
    dv = jnp.tanh(dinv[...]) if tanh_dinv else dinv[...]
    if c_total > 0:
        parts = [vch[c] for c in range(c_total)]
        v = parts[0] if c_total == 1 else jnp.concatenate(parts, axis=-1)
        a = (v + u[...]) * dv
    else:
        a = u[...] * dv
    if with_mm:
        acc = jnp.dot(a, w[...], preferred_element_type=jnp.float32) + b[...]
    else:
        acc = a + b[...]
    if relu:
        acc = jnp.maximum(acc, 0.0)
    y[...] = acc
    if with_score:
        pv = p[...]
        nrm = jnp.sqrt(jnp.sum(pv * pv))
        sc[...] = jnp.dot(acc, pv, preferred_element_type=jnp.float32) / nrm


@functools.lru_cache(None)
def _tc_combine_mm(n_p, c_total, w_chunk, f_in, f_out, relu, with_score,
                   with_mm, tanh_dinv=False, bm=512):
    grid = (n_p // bm,)
    in_specs = []
    if c_total > 0:
        in_specs.append(pl.BlockSpec((c_total, bm, w_chunk),
                                     lambda i: (0, i, 0)))
    in_specs.append(pl.BlockSpec((bm, f_in), lambda i: (i, 0)))
    in_specs.append(pl.BlockSpec((bm, 1), lambda i: (i, 0)))
    if with_mm:
        in_specs.append(pl.BlockSpec((f_in, f_out), lambda i: (0, 0)))
    in_specs.append(pl.BlockSpec((1, f_out), lambda i: (0, 0)))
    out_specs = pl.BlockSpec((bm, f_out), lambda i: (i, 0))
    out_shape = jax.ShapeDtypeStruct((n_p, f_out), jnp.float32)
    if with_score:
        in_specs.append(pl.BlockSpec((f_out, 1), lambda i: (0, 0)))
        out_specs = [out_specs, pl.BlockSpec((bm, 1), lambda i: (i, 0))]
        out_shape = [out_shape, jax.ShapeDtypeStruct((n_p, 1), jnp.float32)]
    body = functools.partial(_combine_mm_body, c_total=c_total, relu=relu,
                             with_score=with_score, with_mm=with_mm,
                             tanh_dinv=tanh_dinv)
    return pl.pallas_call(body, grid=grid, in_specs=in_specs,
                          out_specs=out_specs, out_shape=out_shape)


def _threshold_body(sc_ref, o_ref, *, k_sel, n_real, rows):
    s = sc_ref[...]
    key = _sortable_i32(s)
    flat = (lax.broadcasted_iota(jnp.int32, (rows, 128), 0) * 128
            + lax.broadcasted_iota(jnp.int32, (rows, 128), 1))
    key = jnp.where(flat < n_real, key, INT_MIN)
    sign = INT_MIN

    def step(i, t_u):
        kbit = 31 - i
        trial = jnp.bitwise_or(t_u, jnp.left_shift(jnp.int32(1), kbit))
        t_s = jnp.bitwise_xor(trial, sign)
        cnt = jnp.sum((key >= t_s).astype(jnp.int32))
        return jnp.where(cnt >= k_sel, trial, t_u)

    t_u = lax.fori_loop(0, 32, step, jnp.int32(0))
    t_s = jnp.bitwise_xor(t_u, sign)
    cnt_gt = jnp.sum((key > t_s).astype(jnp.int32))
    ties = k_sel - cnt_gt
    row = lax.broadcasted_iota(jnp.int32, (8, 128), 0)
    o_ref[...] = jnp.where(row == 0, t_s, jnp.where(row == 1, ties, 0))


@functools.lru_cache(None)
def _tc_threshold(n_p, k_sel, n_real):
    rows = n_p // 128
    body = functools.partial(_threshold_body, k_sel=k_sel, n_real=n_real,
                             rows=rows)
    return pl.pallas_call(
        body,
        in_specs=[pl.BlockSpec((rows, 128), lambda: (0, 0))],
        out_specs=pl.BlockSpec((8, 128), lambda: (0, 0)),
        out_shape=jax.ShapeDtypeStruct((8, 128), jnp.int32),
        grid=(),
    )


def _uscale_body(*refs, mode):
    if mode == "plain":
        x, dinv, o = refs
        o[...] = x[...] * dinv[...]
    elif mode == "tanh":
        x, dinv, vals, o = refs
        o[...] = x[...] * (dinv[...] * jnp.tanh(vals[...]))
    else:
        x, dinv, nidx, o = refs
        m = (nidx[...] >= 0).astype(jnp.float32)
        o[...] = x[...] * (dinv[...] * m)


@functools.lru_cache(None)
def _tc_uscale(n_p, f, mode, bm=512):
    assert n_p % bm == 0
    grid = (n_p // bm,)
    in_specs = [pl.BlockSpec((bm, f), lambda i: (i, 0)),
                pl.BlockSpec((bm, 1), lambda i: (i, 0))]
    if mode == "tanh":
        in_specs.append(pl.BlockSpec((bm, 1), lambda i: (i, 0)))
    elif mode == "mask":
        in_specs.append(pl.BlockSpec((bm, 1), lambda i: (i, 0)))
    return pl.pallas_call(
        functools.partial(_uscale_body, mode=mode),
        grid=grid, in_specs=in_specs,
        out_specs=pl.BlockSpec((bm, f), lambda i: (i, 0)),
        out_shape=jax.ShapeDtypeStruct((n_p, f), jnp.float32),
    )


def _dinv_col(deg2, n_p):
    return lax.rsqrt(deg2[:n_p] + deg2[n_p:] + 1.0).reshape(-1, 1)


def _segsum(u, src, dst, n_p, c_total, w):
    u2d = u.reshape(n_p * c_total, w)
    return _sc_segsum(n_p, EP, c_total, w)(u2d, src, dst.reshape(EP // 128, 128))


def _pool(x_lvl, score, src, dst, n_p, k_p, n_real, k_real, n_old_p):
    sc_flat = score.reshape(n_p)
    thr = _tc_threshold(n_p, k_real, n_real)(score.reshape(n_p // 128, 128))
    perm, nidx = _sc_compact(n_p, k_p, n_real, k_real)(sc_flat, thr)
    ns, nd, deg2 = _sc_relabel(n_p, k_p, k_real, EP)(src, dst, nidx)
    xp, vals = _sc_gather_rows(k_p, n_p, HID, True, False)(x_lvl, perm, sc_flat)
    return xp, vals, ns, nd, _dinv_col(deg2, k_p), nidx


def kernel(x, edge_index, batch, W1, b1, W2, b2, W3, b3, W4, b4, W5, b5,
           W6, b6, p1, p2):
    src0 = jnp.concatenate([edge_index[0].astype(jnp.int32),
                            jnp.zeros((EP - E0,), jnp.int32)])
    dst0 = jnp.concatenate([edge_index[1].astype(jnp.int32),
                            jnp.full((EP - E0,), N0, jnp.int32)])
    xp = jnp.concatenate([x, jnp.zeros((N0P - N0, F_IN), jnp.float32)])

    deg0 = _sc_deg0(N0P, EP)(dst0)
    dinv0 = _dinv_col(deg0, N0P)

    ones0 = jnp.ones((N0P, 1), jnp.float32)
    zbh = jnp.zeros((1, HID), jnp.float32)

    xw1 = _tc_combine_mm(N0P, 0, 0, F_IN, HID, False, False, True)(
        xp, ones0, W1, zbh)
    u0 = _tc_uscale(N0P, HID, "plain")(xw1, dinv0)
    v0 = _segsum(u0, src0, dst0, N0P, 8, 128)
    x1, s1 = _tc_combine_mm(N0P, 8, 128, HID, HID, True, True, False)(
        v0, u0, dinv0, b1.reshape(1, HID), p1.reshape(HID, 1))

    x1p, vals1, ns1, nd1, dinv1, nidx1 = _pool(
        x1, s1, src0, dst0, N0P, N1P, N0, K1, N0P)

    xw2 = _tc_combine_mm(N1P, 0, 0, HID, HID, False, False, True,
                         tanh_dinv=True)(x1p, vals1.reshape(N1P, 1), W2, zbh)
    u1 = _tc_uscale(N1P, HID, "plain")(xw2, dinv1)
    v1 = _segsum(u1, ns1, nd1, N1P, 8, 128)
    x2, s2 = _tc_combine_mm(N1P, 8, 128, HID, HID, True, True, False)(
        v1, u1, dinv1, b2.reshape(1, HID), p2.reshape(HID, 1))

    x2p, vals2, ns2, nd2, dinv2, nidx2 = _pool(
        x2, s2, ns1, nd1, N1P, N2P, K1, K2, N1P)

    u2 = _tc_uscale(N2P, HID, "tanh")(x2p, dinv2, vals2.reshape(N2P, 1))
    v2 = _segsum(u2, ns2, nd2, N2P, 8, 128)
    x3 = _tc_combine_mm(N2P, 8, 128, HID, HID, True, False, True)(
        v2, u2, dinv2, W3, b3.reshape(1, HID))

    h1 = _sc_gather_rows(N1P, N2P, HID, False, True)(x3, nidx2)
    u1b = _tc_uscale(N1P, HID, "mask")(h1, dinv1, nidx2.reshape(N1P, 1))
    v1b = _segsum(u1b, ns1, nd1, N1P, 8, 128)
    x4 = _tc_combine_mm(N1P, 8, 128, HID, HID, True, False, True)(
        v1b, u1b, dinv1, W4, b4.reshape(1, HID))

    h0 = _sc_gather_rows(N0P, N1P, HID, False, True)(x4, nidx1)
    u0b = _tc_uscale(N0P, HID, "mask")(h0, dinv0, nidx1.reshape(N0P, 1))
    v0b = _segsum(u0b, src0, dst0, N0P, 8, 128)
    x5 = _tc_combine_mm(N0P, 8, 128, HID, HID, True, False, True)(
        v0b, u0b, dinv0, W5, b5.reshape(1, HID))

    u5 = _tc_uscale(N0P, HID, "plain")(x5, dinv0)
    zb = jnp.zeros((1, F_OUT), jnp.float32)
    w6 = _tc_combine_mm(N0P, 0, 0, HID, F_OUT, False, False, True)(
        u5, dinv0 * 0.0 + 1.0, W6, zb)
    v6 = _segsum(w6, src0, dst0, N0P, 2, 128)
    out = _tc_combine_mm(N0P, 2, 128, F_OUT, F_OUT, False, False, False)(
        v6, w6, dinv0, b6.reshape(1, F_OUT))
    return out[:N0]
